# baseline (device time: 77263 ns/iter reference)
import jax
import jax.numpy as jnp
from jax import lax
from jax.experimental import pallas as pl
from jax.experimental.pallas import tpu as pltpu

N_DEV = 16
N_EXPERTS = 32
CAPACITY = 102.0


def kernel(x, router_W, route_idx, expert_W):
    m, d = x.shape
    e_loc, _, h = expert_W.shape

    def body(x_ref, rw_ref, ri_ref, ew_ref, out_ref,
             ew_comm, ri_comm, wsend, wrecv, rsend, rrecv):
        my = lax.axis_index("i")
        left = lax.rem(my - 1 + N_DEV, N_DEV)
        right = lax.rem(my + 1, N_DEV)

        bar = pltpu.get_barrier_semaphore()
        for nbr in (left, right):
            pl.semaphore_signal(
                bar, inc=1,
                device_id=(nbr,), device_id_type=pl.DeviceIdType.MESH,
            )
        pl.semaphore_wait(bar, 2)

        xb = x_ref[:].astype(jnp.bfloat16)
        route = ri_ref[:]

        ew_comm[0] = ew_ref[:].astype(jnp.bfloat16)
        ri_comm[0] = route

        ids = lax.broadcasted_iota(jnp.int32, (1, N_EXPERTS), 1)

        def chunk_contrib(w2, origin):
            e0 = e_loc * origin
            m0 = (route == e0).astype(jnp.bfloat16)
            m1 = (route == e0 + 1).astype(jnp.bfloat16)
            xm = jnp.concatenate([xb * m0, xb * m1], axis=1)
            w = w2.reshape(e_loc * d, h)
            return jnp.dot(xm, w, preferred_element_type=jnp.float32)

        def hist(rchunk):
            oh = (rchunk == ids).astype(jnp.float32)
            return jnp.sum(oh, axis=0, keepdims=True)

        acc = chunk_contrib(ew_comm[0], my)
        prefix = jnp.zeros((1, N_EXPERTS), jnp.float32)

        for hop in range(N_DEV - 1):
            w_rdma = pltpu.make_async_remote_copy(
                src_ref=ew_comm.at[hop],
                dst_ref=ew_comm.at[hop + 1],
                send_sem=wsend.at[hop],
                recv_sem=wrecv.at[hop],
                device_id=(right,),
                device_id_type=pl.DeviceIdType.MESH,
            )
            r_rdma = pltpu.make_async_remote_copy(
                src_ref=ri_comm.at[hop],
                dst_ref=ri_comm.at[hop + 1],
                send_sem=rsend.at[hop],
                recv_sem=rrecv.at[hop],
                device_id=(right,),
                device_id_type=pl.DeviceIdType.MESH,
            )
            w_rdma.start()
            r_rdma.start()
            w_rdma.wait()
            r_rdma.wait()

            origin = lax.rem(my - hop - 1 + N_DEV, N_DEV)
            acc = acc + chunk_contrib(ew_comm[hop + 1], origin)
            prefix = prefix + jnp.where(
                origin < my, hist(ri_comm[hop + 1]), 0.0
            )

        oh_local = (route == ids).astype(jnp.float32)
        row = lax.broadcasted_iota(jnp.int32, (m, m), 0)
        col = lax.broadcasted_iota(jnp.int32, (m, m), 1)
        tril = (row > col).astype(jnp.float32)
        excl = jnp.dot(tril, oh_local, preferred_element_type=jnp.float32)
        before = jnp.sum(
            oh_local * (excl + prefix), axis=1, keepdims=True
        )
        keep = (before < CAPACITY).astype(jnp.float32)
        out_ref[:] = acc * keep

    return pl.pallas_call(
        body,
        out_shape=jax.ShapeDtypeStruct((m, h), jnp.float32),
        in_specs=[pl.BlockSpec(memory_space=pltpu.VMEM)] * 4,
        out_specs=pl.BlockSpec(memory_space=pltpu.VMEM),
        scratch_shapes=[
            pltpu.VMEM((N_DEV, e_loc, d, h), jnp.bfloat16),
            pltpu.VMEM((N_DEV, m, 1), jnp.int32),
            pltpu.SemaphoreType.DMA((N_DEV - 1,)),
            pltpu.SemaphoreType.DMA((N_DEV - 1,)),
            pltpu.SemaphoreType.DMA((N_DEV - 1,)),
            pltpu.SemaphoreType.DMA((N_DEV - 1,)),
        ],
        compiler_params=pltpu.CompilerParams(collective_id=0),
    )(x, router_W, route_idx, expert_W)


# device time: 76616 ns/iter; 1.0084x vs baseline; 1.0084x over previous
import jax
import jax.numpy as jnp
from jax import lax
from jax.experimental import pallas as pl
from jax.experimental.pallas import tpu as pltpu

N_DEV = 16
N_EXPERTS = 32
CAPACITY = 102.0


def kernel(x, router_W, route_idx, expert_W):
    m, d = x.shape
    e_loc, _, h = expert_W.shape

    def body(x_ref, rw_ref, ri_ref, ew_ref, out_ref,
             ew_comm, ri_comm, wsend, wrecv, rsend, rrecv):
        my = lax.axis_index("i")
        left = lax.rem(my - 1 + N_DEV, N_DEV)
        right = lax.rem(my + 1, N_DEV)

        bar = pltpu.get_barrier_semaphore()
        for nbr in (left, right):
            pl.semaphore_signal(
                bar, inc=1,
                device_id=(nbr,), device_id_type=pl.DeviceIdType.MESH,
            )
        pl.semaphore_wait(bar, 2)

        xb = x_ref[:].astype(jnp.bfloat16)
        route = ri_ref[:]

        ew_comm[0] = ew_ref[:].astype(jnp.bfloat16)
        ri_comm[0] = route

        ids = lax.broadcasted_iota(jnp.int32, (1, N_EXPERTS), 1)

        def chunk_contrib(w2, origin):
            e0 = e_loc * origin
            m0 = (route == e0).astype(jnp.bfloat16)
            m1 = (route == e0 + 1).astype(jnp.bfloat16)
            xm = jnp.concatenate([xb * m0, xb * m1], axis=1)
            w = w2.reshape(e_loc * d, h)
            return jnp.dot(xm, w, preferred_element_type=jnp.float32)

        def hist(rchunk):
            oh = (rchunk == ids).astype(jnp.float32)
            return jnp.sum(oh, axis=0, keepdims=True)

        def mk_w(hop):
            return pltpu.make_async_remote_copy(
                src_ref=ew_comm.at[hop],
                dst_ref=ew_comm.at[hop + 1],
                send_sem=wsend.at[hop],
                recv_sem=wrecv.at[hop],
                device_id=(right,),
                device_id_type=pl.DeviceIdType.MESH,
            )

        def mk_r(hop):
            return pltpu.make_async_remote_copy(
                src_ref=ri_comm.at[hop],
                dst_ref=ri_comm.at[hop + 1],
                send_sem=rsend.at[hop],
                recv_sem=rrecv.at[hop],
                device_id=(right,),
                device_id_type=pl.DeviceIdType.MESH,
            )

        mk_w(0).start()
        mk_r(0).start()

        acc = chunk_contrib(ew_comm[0], my)
        prefix = jnp.zeros((1, N_EXPERTS), jnp.float32)

        for hop in range(N_DEV - 1):
            mk_w(hop).wait_recv()
            mk_r(hop).wait_recv()
            if hop < N_DEV - 2:
                mk_w(hop + 1).start()
                mk_r(hop + 1).start()

            origin = lax.rem(my - hop - 1 + N_DEV, N_DEV)
            acc = acc + chunk_contrib(ew_comm[hop + 1], origin)
            prefix = prefix + jnp.where(
                origin < my, hist(ri_comm[hop + 1]), 0.0
            )

        for hop in range(N_DEV - 1):
            mk_w(hop).wait_send()
            mk_r(hop).wait_send()

        oh_local = (route == ids).astype(jnp.float32)
        row = lax.broadcasted_iota(jnp.int32, (m, m), 0)
        col = lax.broadcasted_iota(jnp.int32, (m, m), 1)
        tril = (row > col).astype(jnp.float32)
        excl = jnp.dot(tril, oh_local, preferred_element_type=jnp.float32)
        before = jnp.sum(
            oh_local * (excl + prefix), axis=1, keepdims=True
        )
        keep = (before < CAPACITY).astype(jnp.float32)
        out_ref[:] = acc * keep

    return pl.pallas_call(
        body,
        out_shape=jax.ShapeDtypeStruct((m, h), jnp.float32),
        in_specs=[pl.BlockSpec(memory_space=pltpu.VMEM)] * 4,
        out_specs=pl.BlockSpec(memory_space=pltpu.VMEM),
        scratch_shapes=[
            pltpu.VMEM((N_DEV, e_loc, d, h), jnp.bfloat16),
            pltpu.VMEM((N_DEV, m, 1), jnp.int32),
            pltpu.SemaphoreType.DMA((N_DEV - 1,)),
            pltpu.SemaphoreType.DMA((N_DEV - 1,)),
            pltpu.SemaphoreType.DMA((N_DEV - 1,)),
            pltpu.SemaphoreType.DMA((N_DEV - 1,)),
        ],
        compiler_params=pltpu.CompilerParams(collective_id=0),
    )(x, router_W, route_idx, expert_W)


# device time: 45150 ns/iter; 1.7113x vs baseline; 1.6969x over previous
import jax
import jax.numpy as jnp
from jax import lax
from jax.experimental import pallas as pl
from jax.experimental.pallas import tpu as pltpu

N_DEV = 16
N_EXPERTS = 32
CAPACITY = 102.0
CW_HOPS = N_DEV // 2
CCW_HOPS = N_DEV - 1 - CW_HOPS


def kernel(x, router_W, route_idx, expert_W):
    m, d = x.shape
    e_loc, _, h = expert_W.shape

    def body(x_ref, rw_ref, ri_ref, ew_ref, out_ref,
             cw_ew, cw_ri, ccw_ew, ccw_ri,
             cw_wsend, cw_wrecv, cw_rsend, cw_rrecv,
             ccw_wsend, ccw_wrecv, ccw_rsend, ccw_rrecv):
        my = lax.axis_index("i")
        left = lax.rem(my - 1 + N_DEV, N_DEV)
        right = lax.rem(my + 1, N_DEV)

        bar = pltpu.get_barrier_semaphore()
        for nbr in (left, right):
            pl.semaphore_signal(
                bar, inc=1,
                device_id=(nbr,), device_id_type=pl.DeviceIdType.MESH,
            )
        pl.semaphore_wait(bar, 2)

        xb = x_ref[:].astype(jnp.bfloat16)
        route = ri_ref[:]

        my_chunk = ew_ref[:].astype(jnp.bfloat16)
        cw_ew[0] = my_chunk
        ccw_ew[0] = my_chunk
        cw_ri[0] = route
        ccw_ri[0] = route

        ids = lax.broadcasted_iota(jnp.int32, (1, N_EXPERTS), 1)

        def chunk_contrib(w2, origin):
            e0 = e_loc * origin
            m0 = (route == e0).astype(jnp.bfloat16)
            m1 = (route == e0 + 1).astype(jnp.bfloat16)
            xm = jnp.concatenate([xb * m0, xb * m1], axis=1)
            w = w2.reshape(e_loc * d, h)
            return jnp.dot(xm, w, preferred_element_type=jnp.float32)

        def hist(rchunk):
            oh = (rchunk == ids).astype(jnp.float32)
            return jnp.sum(oh, axis=0, keepdims=True)

        def mk(ew_buf, ri_buf, wsnd, wrcv, rsnd, rrcv, tgt, hop):
            kw = dict(device_id=(tgt,), device_id_type=pl.DeviceIdType.MESH)
            return (
                pltpu.make_async_remote_copy(
                    src_ref=ew_buf.at[hop], dst_ref=ew_buf.at[hop + 1],
                    send_sem=wsnd.at[hop], recv_sem=wrcv.at[hop], **kw),
                pltpu.make_async_remote_copy(
                    src_ref=ri_buf.at[hop], dst_ref=ri_buf.at[hop + 1],
                    send_sem=rsnd.at[hop], recv_sem=rrcv.at[hop], **kw),
            )

        def mk_cw(hop):
            return mk(cw_ew, cw_ri, cw_wsend, cw_wrecv, cw_rsend, cw_rrecv,
                      right, hop)

        def mk_ccw(hop):
            return mk(ccw_ew, ccw_ri, ccw_wsend, ccw_wrecv, ccw_rsend,
                      ccw_rrecv, left, hop)

        for r_ in mk_cw(0) + mk_ccw(0):
            r_.start()

        acc = chunk_contrib(my_chunk, my)
        prefix = jnp.zeros((1, N_EXPERTS), jnp.float32)

        def absorb(slot_val, route_val, origin):
            c = chunk_contrib(slot_val, origin)
            p = jnp.where(origin < my, hist(route_val), 0.0)
            return c, p

        for hop in range(CW_HOPS):
            wr, rr = mk_cw(hop)
            wr.wait_recv()
            rr.wait_recv()
            if hop + 1 < CW_HOPS:
                for r_ in mk_cw(hop + 1):
                    r_.start()
            cw_origin = lax.rem(my - hop - 1 + N_DEV, N_DEV)

            if hop < CCW_HOPS:
                wr2, rr2 = mk_ccw(hop)
                wr2.wait_recv()
                rr2.wait_recv()
                if hop + 1 < CCW_HOPS:
                    for r_ in mk_ccw(hop + 1):
                        r_.start()

            c, p = absorb(cw_ew[hop + 1], cw_ri[hop + 1], cw_origin)
            acc, prefix = acc + c, prefix + p
            if hop < CCW_HOPS:
                ccw_origin = lax.rem(my + hop + 1, N_DEV)
                c, p = absorb(ccw_ew[hop + 1], ccw_ri[hop + 1], ccw_origin)
                acc, prefix = acc + c, prefix + p

        for hop in range(CW_HOPS):
            for r_ in mk_cw(hop):
                r_.wait_send()
        for hop in range(CCW_HOPS):
            for r_ in mk_ccw(hop):
                r_.wait_send()

        oh_local = (route == ids).astype(jnp.float32)
        row = lax.broadcasted_iota(jnp.int32, (m, m), 0)
        col = lax.broadcasted_iota(jnp.int32, (m, m), 1)
        tril = (row > col).astype(jnp.float32)
        excl = jnp.dot(tril, oh_local, preferred_element_type=jnp.float32)
        before = jnp.sum(
            oh_local * (excl + prefix), axis=1, keepdims=True
        )
        keep = (before < CAPACITY).astype(jnp.float32)
        out_ref[:] = acc * keep

    return pl.pallas_call(
        body,
        out_shape=jax.ShapeDtypeStruct((m, h), jnp.float32),
        in_specs=[pl.BlockSpec(memory_space=pltpu.VMEM)] * 4,
        out_specs=pl.BlockSpec(memory_space=pltpu.VMEM),
        scratch_shapes=[
            pltpu.VMEM((CW_HOPS + 1, e_loc, d, h), jnp.bfloat16),
            pltpu.VMEM((CW_HOPS + 1, m, 1), jnp.int32),
            pltpu.VMEM((CCW_HOPS + 1, e_loc, d, h), jnp.bfloat16),
            pltpu.VMEM((CCW_HOPS + 1, m, 1), jnp.int32),
            pltpu.SemaphoreType.DMA((CW_HOPS,)),
            pltpu.SemaphoreType.DMA((CW_HOPS,)),
            pltpu.SemaphoreType.DMA((CW_HOPS,)),
            pltpu.SemaphoreType.DMA((CW_HOPS,)),
            pltpu.SemaphoreType.DMA((CCW_HOPS,)),
            pltpu.SemaphoreType.DMA((CCW_HOPS,)),
            pltpu.SemaphoreType.DMA((CCW_HOPS,)),
            pltpu.SemaphoreType.DMA((CCW_HOPS,)),
        ],
        compiler_params=pltpu.CompilerParams(collective_id=0),
    )(x, router_W, route_idx, expert_W)
